# baseline (speedup 1.0000x reference)
import jax
import jax.numpy as jnp
from jax.experimental import pallas as pl
from jax.experimental.pallas import tpu as pltpu

_N = 10000
_BM = 200
_R = _N // _BM
_CH = 1024
_NCH = 10
_ADJ_SCALE = 4096.0
_P2_SCALE = 4096.0

_IMAX = [min((_CH * (t + 1) - 1) // _BM, _R - 1) for t in range(_NCH)]


def _xw1_body(x_ref, w1_ref, out_ref):
    xb = x_ref[...].astype(jnp.bfloat16)
    out_ref[...] = jnp.dot(
        xb, w1_ref[...], preferred_element_type=jnp.float32
    ).astype(jnp.bfloat16)


def _pass_a_body(adj_ref, p1_ref, b1_ref, w2_ref, p2_ref, part_ref,
                 *chunk_refs_and_scr):
    chunk_refs = chunk_refs_and_scr[:_NCH]
    p2_scr = chunk_refs_and_scr[_NCH]
    i = pl.program_id(0)

    @pl.when(i == 0)
    def _init():
        p2_scr[...] = jnp.zeros_like(p2_scr)

    a = adj_ref[...].astype(jnp.bfloat16)
    acc = jnp.dot(a, p1_ref[...], preferred_element_type=jnp.float32)
    po = jnp.dot(a, p2_scr[...].astype(jnp.bfloat16),
                 preferred_element_type=jnp.float32)
    a8 = (a * jnp.bfloat16(_ADJ_SCALE)).astype(jnp.float8_e4m3fn)
    for t in range(_NCH):
        hi = min((t + 1) * _CH, _N)
        w = hi - t * _CH

        @pl.when(i <= _IMAX[t])
        def _store(t=t, hi=hi, w=w):
            if w == _CH:
                chunk_refs[t][...] = a8[:, t * _CH:hi]
            else:
                c9 = jnp.pad(
                    a[:, t * _CH:hi] * jnp.bfloat16(_ADJ_SCALE),
                    ((0, 0), (0, _CH - w)),
                )
                chunk_refs[t][...] = c9.astype(jnp.float8_e4m3fn)

    part_ref[...] = po
    h = jnp.maximum(acc + b1_ref[...], 0.0).astype(jnp.bfloat16)
    p2blk = jnp.dot(h, w2_ref[...], preferred_element_type=jnp.float32)
    p2_ref[...] = p2blk
    ext = jnp.concatenate(
        [p2blk, jnp.ones((_BM, 1), jnp.float32),
         jnp.zeros((_BM, 63), jnp.float32)],
        axis=1,
    )
    p2_scr[pl.ds(i * _BM, _BM), :] = ext


def _center_body(p2_ref, c_ref, p2c_ref):
    p2 = p2_ref[...].astype(jnp.float32)
    c = jnp.mean(p2, axis=0, keepdims=True)
    c_ref[...] = c
    cen = jnp.pad((p2 - c) * _P2_SCALE, ((0, _NCH * _CH - _N), (0, 0)))
    p2c_ref[...] = cen.astype(jnp.float8_e4m3fn)


def _pass_b_body(*refs):
    chunk_refs = refs[:_NCH]
    p2c_ref, part_ref, c_ref, b2_ref, out_ref = refs[_NCH:]
    i = pl.program_id(0)
    r0 = i * _BM

    upacc = jnp.zeros((_BM, 64), jnp.float32)
    rowid = jax.lax.broadcasted_iota(jnp.int32, (_CH, 64), 0)
    for t in range(_NCH):
        pc = p2c_ref[t * _CH:(t + 1) * _CH, :]
        pcm = jnp.where(t * _CH + rowid >= r0, pc, jnp.zeros_like(pc))
        upacc += jnp.dot(
            chunk_refs[t][...], pcm, preferred_element_type=jnp.float32
        )

    po = part_ref[...]
    rlow = po[:, 64:65]
    o = (
        po[:, 0:64]
        + upacc * (1.0 / (_ADJ_SCALE * _P2_SCALE))
        + (1.0 - rlow) * c_ref[...]
        + b2_ref[...]
    )
    m = jnp.max(o, axis=1, keepdims=True)
    lse = jnp.log(jnp.sum(jnp.exp(o - m), axis=1, keepdims=True)) + m
    out_ref[...] = o - lse


def kernel(x, adj, W1, b1, W2, b2):
    n, f = x.shape
    h = W1.shape[1]
    c = W2.shape[1]

    w1b = W1.astype(jnp.bfloat16)
    w2b = W2.astype(jnp.bfloat16)
    b1r = b1.reshape(1, h)
    b2r = b2.reshape(1, c)

    p1 = pl.pallas_call(
        _xw1_body,
        grid=(_R,),
        in_specs=[
            pl.BlockSpec((_BM, f), lambda i: (i, 0)),
            pl.BlockSpec((f, h), lambda i: (0, 0)),
        ],
        out_specs=pl.BlockSpec((_BM, h), lambda i: (i, 0)),
        out_shape=jax.ShapeDtypeStruct((n, h), jnp.bfloat16),
    )(x, w1b)

    def _chunk_map(t):
        return lambda i: (jnp.minimum(i, _IMAX[t]), 0)

    outs_a = pl.pallas_call(
        _pass_a_body,
        grid=(_R,),
        in_specs=[
            pl.BlockSpec((_BM, n), lambda i: (i, 0)),
            pl.BlockSpec((n, h), lambda i: (0, 0)),
            pl.BlockSpec((1, h), lambda i: (0, 0)),
            pl.BlockSpec((h, c), lambda i: (0, 0)),
        ],
        out_specs=[
            pl.BlockSpec((_BM, c), lambda i: (i, 0)),
            pl.BlockSpec((_BM, 128), lambda i: (i, 0)),
        ] + [pl.BlockSpec((_BM, _CH), _chunk_map(t)) for t in range(_NCH)],
        out_shape=[
            jax.ShapeDtypeStruct((n, c), jnp.float32),
            jax.ShapeDtypeStruct((n, 128), jnp.float32),
        ] + [
            jax.ShapeDtypeStruct((n, _CH), jnp.float8_e4m3fn)
            for _ in range(_NCH)
        ],
        scratch_shapes=[pltpu.VMEM((n, 128), jnp.float32)],
        compiler_params=pltpu.CompilerParams(
            dimension_semantics=("arbitrary",),
        ),
    )(adj, p1, b1r, w2b)
    p2, part = outs_a[0], outs_a[1]
    chunks = outs_a[2:]

    cmean, p2c = pl.pallas_call(
        _center_body,
        out_shape=[
            jax.ShapeDtypeStruct((1, c), jnp.float32),
            jax.ShapeDtypeStruct((_NCH * _CH, c), jnp.float8_e4m3fn),
        ],
    )(p2)

    out = pl.pallas_call(
        _pass_b_body,
        grid=(_R,),
        in_specs=[
            pl.BlockSpec((_BM, _CH), _chunk_map(t)) for t in range(_NCH)
        ] + [
            pl.BlockSpec((_NCH * _CH, c), lambda i: (0, 0)),
            pl.BlockSpec((_BM, 128), lambda i: (i, 0)),
            pl.BlockSpec((1, c), lambda i: (0, 0)),
            pl.BlockSpec((1, c), lambda i: (0, 0)),
        ],
        out_specs=pl.BlockSpec((_BM, c), lambda i: (i, 0)),
        out_shape=jax.ShapeDtypeStruct((n, c), jnp.float32),
    )(*chunks, p2c, part, cmean, b2r)
    return out


# speedup vs baseline: 1.7428x; 1.7428x over previous
import jax
import jax.numpy as jnp
from jax.experimental import pallas as pl
from jax.experimental.pallas import tpu as pltpu

_BM = 400


_ADJ_SCALE = 8192.0
_P2_SCALE = 1024.0


def _xw1_body(x_ref, w1_ref, out_ref):
    xb = x_ref[...].astype(jnp.bfloat16)
    out_ref[...] = jnp.dot(
        xb, w1_ref[...], preferred_element_type=jnp.float32
    ).astype(jnp.bfloat16)


def _layer1_body(adj_ref, p1_ref, b1_ref, w2_ref, out_ref, adj8_ref):
    a32 = adj_ref[...]
    adj8_ref[...] = (a32 * _ADJ_SCALE).astype(jnp.float4_e2m1fn)
    a = a32.astype(jnp.bfloat16)
    acc = jnp.dot(a, p1_ref[...], preferred_element_type=jnp.float32)
    h = jnp.maximum(acc + b1_ref[...], 0.0).astype(jnp.bfloat16)
    out_ref[...] = jnp.dot(
        h, w2_ref[...], preferred_element_type=jnp.float32
    ).astype(jnp.bfloat16)


def _center_body(p2_ref, c_ref, p2c_ref):
    p2 = p2_ref[...].astype(jnp.float32)
    c = jnp.mean(p2, axis=0, keepdims=True)
    c_ref[...] = c
    p2c_ref[...] = ((p2 - c) * _P2_SCALE).astype(jnp.float4_e2m1fn)


def _layer2_body(adj8_ref, p2c_ref, c_ref, b2_ref, out_ref):
    o = jnp.dot(
        adj8_ref[...], p2c_ref[...], preferred_element_type=jnp.float32
    ) * (1.0 / (_ADJ_SCALE * _P2_SCALE)) + (c_ref[...] + b2_ref[...])
    m = jnp.max(o, axis=1, keepdims=True)
    lse = jnp.log(jnp.sum(jnp.exp(o - m), axis=1, keepdims=True)) + m
    out_ref[...] = o - lse


def kernel(x, adj, W1, b1, W2, b2):
    n, f = x.shape
    h = W1.shape[1]
    c = W2.shape[1]
    bm = _BM
    grid = (n // bm,)

    w1b = W1.astype(jnp.bfloat16)
    w2b = W2.astype(jnp.bfloat16)
    b1r = b1.reshape(1, h)
    b2r = b2.reshape(1, c)

    p1 = pl.pallas_call(
        _xw1_body,
        grid=grid,
        in_specs=[
            pl.BlockSpec((bm, f), lambda i: (i, 0)),
            pl.BlockSpec((f, h), lambda i: (0, 0)),
        ],
        out_specs=pl.BlockSpec((bm, h), lambda i: (i, 0)),
        out_shape=jax.ShapeDtypeStruct((n, h), jnp.bfloat16),
    )(x, w1b)

    p2, adj8 = pl.pallas_call(
        _layer1_body,
        grid=grid,
        in_specs=[
            pl.BlockSpec((bm, n), lambda i: (i, 0)),
            pl.BlockSpec((n, h), lambda i: (0, 0)),
            pl.BlockSpec((1, h), lambda i: (0, 0)),
            pl.BlockSpec((h, c), lambda i: (0, 0)),
        ],
        out_specs=[
            pl.BlockSpec((bm, c), lambda i: (i, 0)),
            pl.BlockSpec((bm, n), lambda i: (i, 0)),
        ],
        out_shape=[
            jax.ShapeDtypeStruct((n, c), jnp.bfloat16),
            jax.ShapeDtypeStruct((n, n), jnp.float4_e2m1fn),
        ],
    )(adj, p1, b1r, w2b)

    cmean, p2c = pl.pallas_call(
        _center_body,
        out_shape=[
            jax.ShapeDtypeStruct((1, c), jnp.float32),
            jax.ShapeDtypeStruct((n, c), jnp.float4_e2m1fn),
        ],
    )(p2)

    out = pl.pallas_call(
        _layer2_body,
        grid=grid,
        in_specs=[
            pl.BlockSpec((bm, n), lambda i: (i, 0)),
            pl.BlockSpec((n, c), lambda i: (0, 0)),
            pl.BlockSpec((1, c), lambda i: (0, 0)),
            pl.BlockSpec((1, c), lambda i: (0, 0)),
        ],
        out_specs=pl.BlockSpec((bm, c), lambda i: (i, 0)),
        out_shape=jax.ShapeDtypeStruct((n, c), jnp.float32),
    )(adj8, p2c, cmean, b2r)
    return out


# speedup vs baseline: 1.8498x; 1.0614x over previous
import jax
import jax.numpy as jnp
from jax.experimental import pallas as pl
from jax.experimental.pallas import tpu as pltpu

_BM = 400


_ADJ_SCALE = 8192.0
_P2_SCALE = 1024.0


def _xw1_body(x_ref, w1_ref, out_ref):
    xb = x_ref[...].astype(jnp.bfloat16)
    out_ref[...] = jnp.dot(
        xb, w1_ref[...], preferred_element_type=jnp.float32
    ).astype(jnp.bfloat16)


def _layer1_body(adj_ref, p1_ref, b1_ref, w2_ref, out_ref, adj8_ref):
    a32 = adj_ref[...]
    adj8_ref[...] = (a32 * _ADJ_SCALE).astype(jnp.float4_e2m1fn)
    a = a32.astype(jnp.bfloat16)
    acc = jnp.dot(a, p1_ref[...], preferred_element_type=jnp.float32)
    h = jnp.maximum(acc + b1_ref[...], 0.0).astype(jnp.bfloat16)
    out_ref[...] = jnp.dot(
        h, w2_ref[...], preferred_element_type=jnp.float32
    ).astype(jnp.bfloat16)


def _layer2_body(adj4_ref, p2_ref, b2_ref, out_ref, c_scr, p2c_scr):
    i = pl.program_id(0)

    @pl.when(i == 0)
    def _center():
        p2 = p2_ref[...].astype(jnp.float32)
        c = jnp.mean(p2, axis=0, keepdims=True)
        c_scr[0:1, :] = c
        p2c_scr[...] = ((p2 - c) * _P2_SCALE).astype(jnp.float8_e4m3fn)

    a8 = adj4_ref[...].astype(jnp.float8_e4m3fn)
    o = jnp.dot(
        a8, p2c_scr[...], preferred_element_type=jnp.float32
    ) * (1.0 / (_ADJ_SCALE * _P2_SCALE)) + (c_scr[0:1, :] + b2_ref[...])
    m = jnp.max(o, axis=1, keepdims=True)
    lse = jnp.log(jnp.sum(jnp.exp(o - m), axis=1, keepdims=True)) + m
    out_ref[...] = o - lse


def kernel(x, adj, W1, b1, W2, b2):
    n, f = x.shape
    h = W1.shape[1]
    c = W2.shape[1]
    bm = _BM
    grid = (n // bm,)

    w1b = W1.astype(jnp.bfloat16)
    w2b = W2.astype(jnp.bfloat16)
    b1r = b1.reshape(1, h)
    b2r = b2.reshape(1, c)

    p1 = pl.pallas_call(
        _xw1_body,
        grid=grid,
        in_specs=[
            pl.BlockSpec((bm, f), lambda i: (i, 0)),
            pl.BlockSpec((f, h), lambda i: (0, 0)),
        ],
        out_specs=pl.BlockSpec((bm, h), lambda i: (i, 0)),
        out_shape=jax.ShapeDtypeStruct((n, h), jnp.bfloat16),
    )(x, w1b)

    p2, adj8 = pl.pallas_call(
        _layer1_body,
        grid=grid,
        in_specs=[
            pl.BlockSpec((bm, n), lambda i: (i, 0)),
            pl.BlockSpec((n, h), lambda i: (0, 0)),
            pl.BlockSpec((1, h), lambda i: (0, 0)),
            pl.BlockSpec((h, c), lambda i: (0, 0)),
        ],
        out_specs=[
            pl.BlockSpec((bm, c), lambda i: (i, 0)),
            pl.BlockSpec((bm, n), lambda i: (i, 0)),
        ],
        out_shape=[
            jax.ShapeDtypeStruct((n, c), jnp.bfloat16),
            jax.ShapeDtypeStruct((n, n), jnp.float4_e2m1fn),
        ],
    )(adj, p1, b1r, w2b)

    out = pl.pallas_call(
        _layer2_body,
        grid=grid,
        in_specs=[
            pl.BlockSpec((bm, n), lambda i: (i, 0)),
            pl.BlockSpec((n, c), lambda i: (0, 0)),
            pl.BlockSpec((1, c), lambda i: (0, 0)),
        ],
        out_specs=pl.BlockSpec((bm, c), lambda i: (i, 0)),
        out_shape=jax.ShapeDtypeStruct((n, c), jnp.float32),
        scratch_shapes=[
            pltpu.VMEM((8, c), jnp.float32),
            pltpu.VMEM((n, c), jnp.float8_e4m3fn),
        ],
        compiler_params=pltpu.CompilerParams(
            dimension_semantics=("arbitrary",),
        ),
    )(adj8, p2, b2r)
    return out
